# baseline (speedup 1.0000x reference)
import jax
import jax.numpy as jnp
from jax.experimental import pallas as pl
from jax.experimental.pallas import tpu as pltpu
from jax.experimental.pallas import tpu_sc as plsc

_B = 1024
_D = 768
_FF = 3072
_DC = _D // 4
_E = 8
_L = 6
_CAP = 160
_S = _E * _CAP

_PARAMS = pltpu.CompilerParams(vmem_limit_bytes=110 * 1024 * 1024)


def _rms_mul(h, w):
    return h * jax.lax.rsqrt(jnp.mean(h * h, axis=-1, keepdims=True) + 1e-6) * w


def _dot(a, b):
    return jnp.dot(a, b, preferred_element_type=jnp.float32)


def _proj_kernel(x_ref, w_ref, b_ref, o_ref):
    o_ref[...] = _dot(x_ref[...], w_ref[...]) + b_ref[...]


def _attn_block(h_ref, g1_ref, wv_ref, wo_ref, g2_ref, o_ref):
    h = h_ref[...]
    n = _rms_mul(h, g1_ref[0])
    hn = h + _dot(_dot(n, wv_ref[0]), wo_ref[0])
    o_ref[...] = hn
    return _rms_mul(hn, g2_ref[0])


def _dense_kernel(h_ref, g1_ref, wv_ref, wo_ref, g2_ref, wi_ref, w2_ref,
                  o_ref, n2_scr, h1_scr):
    t = pl.program_id(0)

    @pl.when(t == 0)
    def _():
        n2 = _attn_block(h_ref, g1_ref, wv_ref, wo_ref, g2_ref, o_ref)
        for k in range(4):
            n2_scr[k] = n2[:, k * _DC:(k + 1) * _DC]

    @pl.when((t >= 1) & (t <= 4))
    def _():
        k = t - 1
        part = _dot(n2_scr[k], wi_ref[0])
        for f in range(4):
            chunk = part[:, f * _D:(f + 1) * _D]

            @pl.when(k == 0)
            def _(chunk=chunk, f=f):
                h1_scr[f] = chunk

            @pl.when(k > 0)
            def _(chunk=chunk, f=f):
                h1_scr[f] += chunk

    @pl.when(t >= 5)
    def _():
        f = t - 5
        h1 = jnp.maximum(h1_scr[f], 0.0)
        o_ref[...] += _dot(h1, w2_ref[0])


def _moe_kernel(h_ref, g1_ref, wv_ref, wo_ref, g2_ref, rw_ref, wia_ref,
                wib_ref, w2a_ref, w2b_ref, o_ref, h2g_ref, slot_ref,
                ein_scr, gs_scr):
    t = pl.program_id(0)

    @pl.when(t == 0)
    def _():
        n2 = _attn_block(h_ref, g1_ref, wv_ref, wo_ref, g2_ref, o_ref)
        logits = _dot(n2, rw_ref[0])
        col = jax.lax.broadcasted_iota(jnp.int32, (_B, _E), 1)
        m = jnp.max(logits, axis=-1, keepdims=True)
        ex = jnp.exp(logits - m)
        probs = ex / jnp.sum(ex, axis=-1, keepdims=True)
        gate = jnp.max(probs, axis=-1, keepdims=True)
        idx = jnp.min(jnp.where(probs == gate, col, _E), axis=-1,
                      keepdims=True)
        onehot = (col == idx).astype(jnp.float32)
        ri = jax.lax.broadcasted_iota(jnp.int32, (_B, _B), 0)
        ci = jax.lax.broadcasted_iota(jnp.int32, (_B, _B), 1)
        tril = (ci < ri).astype(jnp.float32)
        cnt = _dot(tril, onehot)
        pos = jnp.sum(cnt * onehot, axis=-1, keepdims=True).astype(jnp.int32)
        slot = jnp.where(pos < _CAP, idx * _CAP + pos, _S)
        slot_ref[...] = slot
        scol = jax.lax.broadcasted_iota(jnp.int32, (_B, _S), 1)
        pt = (scol == slot).astype(jnp.float32)
        gs_scr[...] = jax.lax.dot_general(
            pt, gate, (((0,), (0,)), ((), ())),
            preferred_element_type=jnp.float32)
        ein_scr[...] = jax.lax.dot_general(
            pt, n2, (((0,), (0,)), ((), ())),
            preferred_element_type=jnp.float32)
        h2g_ref[pl.ds(_S, 16), :] = jnp.zeros((16, _D), jnp.float32)

    @pl.when((t >= 1) & (t <= _E))
    def _():
        e = t - 1
        rows = ein_scr[pl.ds(e * _CAP, _CAP), :]
        h1 = jnp.maximum(
            _dot(rows[:, :_D // 2], wia_ref[0, 0]) +
            _dot(rows[:, _D // 2:], wib_ref[0, 0]), 0.0)
        h2 = (_dot(h1[:, :_FF // 2], w2a_ref[0, 0]) +
              _dot(h1[:, _FF // 2:], w2b_ref[0, 0]))
        h2g_ref[pl.ds(e * _CAP, _CAP), :] = (
            h2 * gs_scr[pl.ds(e * _CAP, _CAP), :])


_NW = 32
_BW = _B // _NW


def _sc_combine(h2g_ref, slot_ref, hatt_ref, out_ref, idx_v, rows_v, hrow_v,
                sem):
    wid = jax.lax.axis_index("s") * 2 + jax.lax.axis_index("c")
    base = wid * _BW
    pltpu.sync_copy(slot_ref.at[pl.ds(base, _BW)], idx_v)
    pltpu.async_copy(h2g_ref.at[idx_v], rows_v, sem).wait()
    pltpu.sync_copy(hatt_ref.at[pl.ds(base, _BW)], hrow_v)

    def row_add(r, carry):
        for c in range(_D // 16):
            sl = pl.ds(c * 16, 16)
            rows_v[r, sl] += hrow_v[r, sl]
        return carry

    jax.lax.fori_loop(0, _BW, row_add, 0)
    pltpu.sync_copy(rows_v, out_ref.at[pl.ds(base, _BW)])


def _final_kernel(h_ref, g_ref, w_ref, b_ref, o_ref):
    n = _rms_mul(h_ref[...], g_ref[...])
    o_ref[...] = _dot(n, w_ref[...]) + b_ref[...]


def kernel(x, proj_W, proj_b, attn_q, attn_k, attn_v, attn_o, ln1, ln2,
           router_W, moe_wi, moe_wo, ffn_wi, ffn_wo, final_ln, fc_W, fc_b):
    f32 = jnp.float32
    sd = jax.ShapeDtypeStruct
    xf = x.reshape(_B, -1)
    ln1r = ln1.reshape(_L, 1, _D)
    ln2r = ln2.reshape(_L, 1, _D)

    h = pl.pallas_call(
        _proj_kernel,
        grid=(4,),
        in_specs=[
            pl.BlockSpec((_B // 4, 3072), lambda k: (k, 0)),
            pl.BlockSpec((3072, _D), lambda k: (0, 0)),
            pl.BlockSpec((1, _D), lambda k: (0, 0)),
        ],
        out_specs=pl.BlockSpec((_B // 4, _D), lambda k: (k, 0)),
        out_shape=sd((_B, _D), f32),
        compiler_params=_PARAMS)(xf, proj_W, proj_b.reshape(1, _D))

    for i in range(_L):
        j = i // 2
        if i % 2 == 1:
            hatt, h2g, slot = pl.pallas_call(
                _moe_kernel,
                grid=(_E + 1,),
                in_specs=[
                    pl.BlockSpec((_B, _D), lambda t: (0, 0)),
                    pl.BlockSpec((1, 1, _D), lambda t, i=i: (i, 0, 0)),
                    pl.BlockSpec((1, _D, _D), lambda t, i=i: (i, 0, 0)),
                    pl.BlockSpec((1, _D, _D), lambda t, i=i: (i, 0, 0)),
                    pl.BlockSpec((1, 1, _D), lambda t, i=i: (i, 0, 0)),
                    pl.BlockSpec((1, _D, _E), lambda t, j=j: (j, 0, 0)),
                    pl.BlockSpec(
                        (1, 1, _D // 2, _FF),
                        lambda t, j=j: (j, jnp.clip(t - 1, 0, _E - 1), 0, 0)),
                    pl.BlockSpec(
                        (1, 1, _D // 2, _FF),
                        lambda t, j=j: (j, jnp.clip(t - 1, 0, _E - 1), 1, 0)),
                    pl.BlockSpec(
                        (1, 1, _FF // 2, _D),
                        lambda t, j=j: (j, jnp.clip(t - 1, 0, _E - 1), 0, 0)),
                    pl.BlockSpec(
                        (1, 1, _FF // 2, _D),
                        lambda t, j=j: (j, jnp.clip(t - 1, 0, _E - 1), 1, 0)),
                ],
                out_specs=[
                    pl.BlockSpec((_B, _D), lambda t: (0, 0)),
                    pl.BlockSpec((_S + 16, _D), lambda t: (0, 0)),
                    pl.BlockSpec((_B, 1), lambda t: (0, 0)),
                ],
                out_shape=[
                    sd((_B, _D), f32),
                    sd((_S + 16, _D), f32),
                    sd((_B, 1), jnp.int32),
                ],
                scratch_shapes=[
                    pltpu.VMEM((_S, _D), f32),
                    pltpu.VMEM((_S, 1), f32),
                ],
                compiler_params=_PARAMS)(
                    h, ln1r, attn_v, attn_o, ln2r, router_W,
                    moe_wi, moe_wi, moe_wo, moe_wo)
            h = pl.kernel(
                _sc_combine,
                out_type=sd((_B, _D), f32),
                mesh=plsc.VectorSubcoreMesh(core_axis_name="c",
                                            subcore_axis_name="s"),
                scratch_types=[
                    pltpu.VMEM((_BW,), jnp.int32),
                    pltpu.VMEM((_BW, _D), f32),
                    pltpu.VMEM((_BW, _D), f32),
                    pltpu.SemaphoreType.DMA,
                ])(h2g, slot.reshape(_B), hatt)
        else:
            h = pl.pallas_call(
                _dense_kernel,
                grid=(9,),
                in_specs=[
                    pl.BlockSpec((_B, _D), lambda t: (0, 0)),
                    pl.BlockSpec((1, 1, _D), lambda t, i=i: (i, 0, 0)),
                    pl.BlockSpec((1, _D, _D), lambda t, i=i: (i, 0, 0)),
                    pl.BlockSpec((1, _D, _D), lambda t, i=i: (i, 0, 0)),
                    pl.BlockSpec((1, 1, _D), lambda t, i=i: (i, 0, 0)),
                    pl.BlockSpec((1, _DC, _FF),
                                 lambda t, j=j: (j, jnp.clip(t - 1, 0, 3), 0)),
                    pl.BlockSpec((1, _D, _D),
                                 lambda t, j=j: (j, jnp.clip(t - 5, 0, 3), 0)),
                ],
                out_specs=pl.BlockSpec((_B, _D), lambda t: (0, 0)),
                out_shape=sd((_B, _D), f32),
                scratch_shapes=[
                    pltpu.VMEM((4, _B, _DC), f32),
                    pltpu.VMEM((4, _B, _D), f32),
                ],
                compiler_params=_PARAMS)(
                    h, ln1r, attn_v, attn_o, ln2r, ffn_wi, ffn_wo)

    out = pl.pallas_call(
        _final_kernel,
        out_shape=sd((_B, 10), f32),
        compiler_params=_PARAMS)(
            h, final_ln.reshape(1, _D), fc_W, fc_b.reshape(1, 10))
    return out


# speedup vs baseline: 1.0123x; 1.0123x over previous
import jax
import jax.numpy as jnp
from jax.experimental import pallas as pl
from jax.experimental.pallas import tpu as pltpu
from jax.experimental.pallas import tpu_sc as plsc

_B = 1024
_D = 768
_FF = 3072
_DC = _D // 4
_E = 8
_L = 6
_CAP = 160
_S = _E * _CAP

_PARAMS = pltpu.CompilerParams(vmem_limit_bytes=110 * 1024 * 1024)


def _rms_mul(h, w):
    return h * jax.lax.rsqrt(jnp.mean(h * h, axis=-1, keepdims=True) + 1e-6) * w


def _dot(a, b):
    return jnp.dot(a, b, preferred_element_type=jnp.float32)


def _proj_kernel(x_ref, w_ref, b_ref, o_ref):
    o_ref[...] = _dot(x_ref[...], w_ref[...]) + b_ref[...]


def _attn_block(h_ref, g1_ref, wv_ref, wo_ref, g2_ref, o_ref, hb_ref=None):
    h = h_ref[...]
    if hb_ref is not None:
        h = h + hb_ref[...]
    n = _rms_mul(h, g1_ref[0])
    hn = h + _dot(_dot(n, wv_ref[0]), wo_ref[0])
    o_ref[...] = hn
    return _rms_mul(hn, g2_ref[0])


def _make_dense_kernel(two_h):
    def _dense_kernel(*args):
        if two_h:
            (h_ref, hb_ref, g1_ref, wv_ref, wo_ref, g2_ref, wi_ref, w2_ref,
             o_ref, n2_scr, h1_scr) = args
        else:
            (h_ref, g1_ref, wv_ref, wo_ref, g2_ref, wi_ref, w2_ref,
             o_ref, n2_scr, h1_scr) = args
            hb_ref = None
        t = pl.program_id(0)

        @pl.when(t == 0)
        def _():
            n2 = _attn_block(h_ref, g1_ref, wv_ref, wo_ref, g2_ref, o_ref,
                             hb_ref)
            for k in range(4):
                n2_scr[k] = n2[:, k * _DC:(k + 1) * _DC]

        @pl.when((t >= 1) & (t <= 4))
        def _():
            k = t - 1
            part = _dot(n2_scr[k], wi_ref[0])
            for f in range(4):
                chunk = part[:, f * _D:(f + 1) * _D]

                @pl.when(k == 0)
                def _(chunk=chunk, f=f):
                    h1_scr[f] = chunk

                @pl.when(k > 0)
                def _(chunk=chunk, f=f):
                    h1_scr[f] += chunk

        @pl.when(t >= 5)
        def _():
            f = t - 5
            h1 = jnp.maximum(h1_scr[f], 0.0)
            o_ref[...] += _dot(h1, w2_ref[0])

    return _dense_kernel


def _moe_kernel(h_ref, g1_ref, wv_ref, wo_ref, g2_ref, rw_ref, wia_ref,
                wib_ref, w2a_ref, w2b_ref, o_ref, h2g_ref, slot_ref,
                ein_scr, gs_scr):
    t = pl.program_id(0)

    @pl.when(t == 0)
    def _():
        n2 = _attn_block(h_ref, g1_ref, wv_ref, wo_ref, g2_ref, o_ref)
        logits = _dot(n2, rw_ref[0])
        col = jax.lax.broadcasted_iota(jnp.int32, (_B, _E), 1)
        m = jnp.max(logits, axis=-1, keepdims=True)
        ex = jnp.exp(logits - m)
        probs = ex / jnp.sum(ex, axis=-1, keepdims=True)
        gate = jnp.max(probs, axis=-1, keepdims=True)
        idx = jnp.min(jnp.where(probs == gate, col, _E), axis=-1,
                      keepdims=True)
        onehot = (col == idx).astype(jnp.float32)
        ri = jax.lax.broadcasted_iota(jnp.int32, (_B, _B), 0)
        ci = jax.lax.broadcasted_iota(jnp.int32, (_B, _B), 1)
        tril = (ci < ri).astype(jnp.float32)
        cnt = _dot(tril, onehot)
        pos = jnp.sum(cnt * onehot, axis=-1, keepdims=True).astype(jnp.int32)
        slot = jnp.where(pos < _CAP, idx * _CAP + pos, _S)
        slot_ref[...] = slot
        scol = jax.lax.broadcasted_iota(jnp.int32, (_B, _S), 1)
        pt = (scol == slot).astype(jnp.float32)
        gs_scr[...] = jax.lax.dot_general(
            pt, gate, (((0,), (0,)), ((), ())),
            preferred_element_type=jnp.float32)
        ein_scr[...] = jax.lax.dot_general(
            pt, n2, (((0,), (0,)), ((), ())),
            preferred_element_type=jnp.float32)
        h2g_ref[pl.ds(_S, 16), :] = jnp.zeros((16, _D), jnp.float32)

    @pl.when((t >= 1) & (t <= _E))
    def _():
        e = t - 1
        rows = ein_scr[pl.ds(e * _CAP, _CAP), :]
        h1 = jnp.maximum(
            _dot(rows[:, :_D // 2], wia_ref[0, 0]) +
            _dot(rows[:, _D // 2:], wib_ref[0, 0]), 0.0)
        h2 = (_dot(h1[:, :_FF // 2], w2a_ref[0, 0]) +
              _dot(h1[:, _FF // 2:], w2b_ref[0, 0]))
        h2g_ref[pl.ds(e * _CAP, _CAP), :] = (
            h2 * gs_scr[pl.ds(e * _CAP, _CAP), :])


_NW = 32
_BW = _B // _NW


def _sc_combine(h2g_ref, slot_ref, out_ref, idx_v, rows_v, sem):
    wid = jax.lax.axis_index("s") * 2 + jax.lax.axis_index("c")
    base = wid * _BW
    pltpu.sync_copy(slot_ref.at[pl.ds(base, _BW)], idx_v)
    pltpu.async_copy(h2g_ref.at[idx_v], rows_v, sem).wait()
    pltpu.sync_copy(rows_v, out_ref.at[pl.ds(base, _BW)])


def _final_kernel(ha_ref, hb_ref, g_ref, w_ref, b_ref, o_ref):
    n = _rms_mul(ha_ref[...] + hb_ref[...], g_ref[...])
    o_ref[...] = _dot(n, w_ref[...]) + b_ref[...]


def kernel(x, proj_W, proj_b, attn_q, attn_k, attn_v, attn_o, ln1, ln2,
           router_W, moe_wi, moe_wo, ffn_wi, ffn_wo, final_ln, fc_W, fc_b):
    f32 = jnp.float32
    sd = jax.ShapeDtypeStruct
    xf = x.reshape(_B, -1)
    ln1r = ln1.reshape(_L, 1, _D)
    ln2r = ln2.reshape(_L, 1, _D)

    h = pl.pallas_call(
        _proj_kernel,
        grid=(4,),
        in_specs=[
            pl.BlockSpec((_B // 4, 3072), lambda k: (k, 0)),
            pl.BlockSpec((3072, _D), lambda k: (0, 0)),
            pl.BlockSpec((1, _D), lambda k: (0, 0)),
        ],
        out_specs=pl.BlockSpec((_B // 4, _D), lambda k: (k, 0)),
        out_shape=sd((_B, _D), f32),
        compiler_params=_PARAMS)(xf, proj_W, proj_b.reshape(1, _D))

    hb = None
    for i in range(_L):
        j = i // 2
        if i % 2 == 1:
            hatt, h2g, slot = pl.pallas_call(
                _moe_kernel,
                grid=(_E + 1,),
                in_specs=[
                    pl.BlockSpec((_B, _D), lambda t: (0, 0)),
                    pl.BlockSpec((1, 1, _D), lambda t, i=i: (i, 0, 0)),
                    pl.BlockSpec((1, _D, _D), lambda t, i=i: (i, 0, 0)),
                    pl.BlockSpec((1, _D, _D), lambda t, i=i: (i, 0, 0)),
                    pl.BlockSpec((1, 1, _D), lambda t, i=i: (i, 0, 0)),
                    pl.BlockSpec((1, _D, _E), lambda t, j=j: (j, 0, 0)),
                    pl.BlockSpec(
                        (1, 1, _D // 2, _FF),
                        lambda t, j=j: (j, jnp.clip(t - 1, 0, _E - 1), 0, 0)),
                    pl.BlockSpec(
                        (1, 1, _D // 2, _FF),
                        lambda t, j=j: (j, jnp.clip(t - 1, 0, _E - 1), 1, 0)),
                    pl.BlockSpec(
                        (1, 1, _FF // 2, _D),
                        lambda t, j=j: (j, jnp.clip(t - 1, 0, _E - 1), 0, 0)),
                    pl.BlockSpec(
                        (1, 1, _FF // 2, _D),
                        lambda t, j=j: (j, jnp.clip(t - 1, 0, _E - 1), 1, 0)),
                ],
                out_specs=[
                    pl.BlockSpec((_B, _D), lambda t: (0, 0)),
                    pl.BlockSpec((_S + 16, _D), lambda t: (0, 0)),
                    pl.BlockSpec((_B, 1), lambda t: (0, 0)),
                ],
                out_shape=[
                    sd((_B, _D), f32),
                    sd((_S + 16, _D), f32),
                    sd((_B, 1), jnp.int32),
                ],
                scratch_shapes=[
                    pltpu.VMEM((_S, _D), f32),
                    pltpu.VMEM((_S, 1), f32),
                ],
                compiler_params=_PARAMS)(
                    h, ln1r, attn_v, attn_o, ln2r, router_W,
                    moe_wi, moe_wi, moe_wo, moe_wo)
            moe = pl.kernel(
                _sc_combine,
                out_type=sd((_B, _D), f32),
                mesh=plsc.VectorSubcoreMesh(core_axis_name="c",
                                            subcore_axis_name="s"),
                scratch_types=[
                    pltpu.VMEM((_BW,), jnp.int32),
                    pltpu.VMEM((_BW, _D), f32),
                    pltpu.SemaphoreType.DMA,
                ])(h2g, slot.reshape(_B))
            h, hb = hatt, moe
        else:
            two_h = hb is not None
            dense_in = [h, hb] if two_h else [h]
            h = pl.pallas_call(
                _make_dense_kernel(two_h),
                grid=(9,),
                in_specs=[pl.BlockSpec((_B, _D), lambda t: (0, 0))] * (
                    2 if two_h else 1) + [
                    pl.BlockSpec((1, 1, _D), lambda t, i=i: (i, 0, 0)),
                    pl.BlockSpec((1, _D, _D), lambda t, i=i: (i, 0, 0)),
                    pl.BlockSpec((1, _D, _D), lambda t, i=i: (i, 0, 0)),
                    pl.BlockSpec((1, 1, _D), lambda t, i=i: (i, 0, 0)),
                    pl.BlockSpec((1, _DC, _FF),
                                 lambda t, j=j: (j, jnp.clip(t - 1, 0, 3), 0)),
                    pl.BlockSpec((1, _D, _D),
                                 lambda t, j=j: (j, jnp.clip(t - 5, 0, 3), 0)),
                ],
                out_specs=pl.BlockSpec((_B, _D), lambda t: (0, 0)),
                out_shape=sd((_B, _D), f32),
                scratch_shapes=[
                    pltpu.VMEM((4, _B, _DC), f32),
                    pltpu.VMEM((4, _B, _D), f32),
                ],
                compiler_params=_PARAMS)(
                    *dense_in, ln1r, attn_v, attn_o, ln2r, ffn_wi, ffn_wo)
            hb = None

    out = pl.pallas_call(
        _final_kernel,
        out_shape=sd((_B, 10), f32),
        compiler_params=_PARAMS)(
            h, hb, final_ln.reshape(1, _D), fc_W, fc_b.reshape(1, 10))
    return out


# speedup vs baseline: 1.2414x; 1.2264x over previous
import jax
import jax.numpy as jnp
from jax.experimental import pallas as pl
from jax.experimental.pallas import tpu as pltpu

_B = 1024
_D = 768
_FF = 3072
_DC = _D // 4
_E = 8
_L = 6
_CAP = 160
_S = _E * _CAP

_PARAMS = pltpu.CompilerParams(vmem_limit_bytes=110 * 1024 * 1024)


def _rms_mul(h, w):
    return h * jax.lax.rsqrt(jnp.mean(h * h, axis=-1, keepdims=True) + 1e-6) * w


def _dot(a, b):
    return jnp.dot(a, b, preferred_element_type=jnp.float32)


def _proj_kernel(x_ref, w_ref, b_ref, o_ref):
    o_ref[...] = _dot(x_ref[...], w_ref[...]) + b_ref[...]


def _attn_block(h_ref, g1_ref, wv_ref, wo_ref, g2_ref, o_ref):
    h = h_ref[...]
    n = _rms_mul(h, g1_ref[0])
    hn = h + _dot(_dot(n, wv_ref[0]), wo_ref[0])
    o_ref[...] = hn
    return _rms_mul(hn, g2_ref[0])


def _dense_kernel(h_ref, g1_ref, wv_ref, wo_ref, g2_ref, wi_ref, w2_ref,
                  o_ref, n2_scr):
    t = pl.program_id(0)

    @pl.when(t == 0)
    def _():
        n2_scr[...] = _attn_block(h_ref, g1_ref, wv_ref, wo_ref, g2_ref,
                                  o_ref)

    @pl.when(t >= 1)
    def _():
        h1 = jnp.maximum(_dot(n2_scr[...], wi_ref[0]), 0.0)
        o_ref[...] += _dot(h1, w2_ref[0])


def _moe_kernel(h_ref, g1_ref, wv_ref, wo_ref, g2_ref, rw_ref, wia_ref,
                wib_ref, w2a_ref, w2b_ref, o_ref, ein_scr, ptg_scr):
    t = pl.program_id(0)

    @pl.when(t == 0)
    def _():
        n2 = _attn_block(h_ref, g1_ref, wv_ref, wo_ref, g2_ref, o_ref)
        logits = _dot(n2, rw_ref[0])
        col = jax.lax.broadcasted_iota(jnp.int32, (_B, _E), 1)
        m = jnp.max(logits, axis=-1, keepdims=True)
        ex = jnp.exp(logits - m)
        probs = ex / jnp.sum(ex, axis=-1, keepdims=True)
        gate = jnp.max(probs, axis=-1, keepdims=True)
        idx = jnp.min(jnp.where(probs == gate, col, _E), axis=-1,
                      keepdims=True)
        onehot = (col == idx).astype(jnp.float32)
        ri = jax.lax.broadcasted_iota(jnp.int32, (_B, _B), 0)
        ci = jax.lax.broadcasted_iota(jnp.int32, (_B, _B), 1)
        tril = (ci < ri).astype(jnp.float32)
        cnt = _dot(tril, onehot)
        pos = jnp.sum(cnt * onehot, axis=-1, keepdims=True).astype(jnp.int32)
        slot = jnp.where(pos < _CAP, idx * _CAP + pos, _S)
        scol = jax.lax.broadcasted_iota(jnp.int32, (_B, _S), 1)
        pt = (scol == slot).astype(jnp.float32)
        ptg_scr[...] = pt * gate
        ein_scr[...] = jax.lax.dot_general(
            pt, n2, (((0,), (0,)), ((), ())),
            preferred_element_type=jnp.float32)

    @pl.when((t >= 1) & (t <= _E))
    def _():
        e = t - 1
        rows = ein_scr[pl.ds(e * _CAP, _CAP), :]
        h1 = jnp.maximum(
            _dot(rows[:, :_D // 2], wia_ref[0, 0]) +
            _dot(rows[:, _D // 2:], wib_ref[0, 0]), 0.0)
        ein_scr[pl.ds(e * _CAP, _CAP), :] = (
            _dot(h1[:, :_FF // 2], w2a_ref[0, 0]) +
            _dot(h1[:, _FF // 2:], w2b_ref[0, 0]))

    @pl.when(t == _E + 1)
    def _():
        o_ref[...] += _dot(ptg_scr[...], ein_scr[...])


def _final_kernel(h_ref, g_ref, w_ref, b_ref, o_ref):
    n = _rms_mul(h_ref[...], g_ref[...])
    o_ref[...] = _dot(n, w_ref[...]) + b_ref[...]


def kernel(x, proj_W, proj_b, attn_q, attn_k, attn_v, attn_o, ln1, ln2,
           router_W, moe_wi, moe_wo, ffn_wi, ffn_wo, final_ln, fc_W, fc_b):
    f32 = jnp.float32
    sd = jax.ShapeDtypeStruct
    xf = x.reshape(_B, -1)
    ln1r = ln1.reshape(_L, 1, _D)
    ln2r = ln2.reshape(_L, 1, _D)

    h = pl.pallas_call(
        _proj_kernel,
        grid=(4,),
        in_specs=[
            pl.BlockSpec((_B // 4, 3072), lambda k: (k, 0)),
            pl.BlockSpec((3072, _D), lambda k: (0, 0)),
            pl.BlockSpec((1, _D), lambda k: (0, 0)),
        ],
        out_specs=pl.BlockSpec((_B // 4, _D), lambda k: (k, 0)),
        out_shape=sd((_B, _D), f32),
        compiler_params=_PARAMS)(xf, proj_W, proj_b.reshape(1, _D))

    for i in range(_L):
        j = i // 2
        if i % 2 == 1:
            h = pl.pallas_call(
                _moe_kernel,
                grid=(_E + 2,),
                in_specs=[
                    pl.BlockSpec((_B, _D), lambda t: (0, 0)),
                    pl.BlockSpec((1, 1, _D), lambda t, i=i: (i, 0, 0)),
                    pl.BlockSpec((1, _D, _D), lambda t, i=i: (i, 0, 0)),
                    pl.BlockSpec((1, _D, _D), lambda t, i=i: (i, 0, 0)),
                    pl.BlockSpec((1, 1, _D), lambda t, i=i: (i, 0, 0)),
                    pl.BlockSpec((1, _D, _E), lambda t, j=j: (j, 0, 0)),
                    pl.BlockSpec(
                        (1, 1, _D // 2, _FF),
                        lambda t, j=j: (j, jnp.clip(t - 1, 0, _E - 1), 0, 0)),
                    pl.BlockSpec(
                        (1, 1, _D // 2, _FF),
                        lambda t, j=j: (j, jnp.clip(t - 1, 0, _E - 1), 1, 0)),
                    pl.BlockSpec(
                        (1, 1, _FF // 2, _D),
                        lambda t, j=j: (j, jnp.clip(t - 1, 0, _E - 1), 0, 0)),
                    pl.BlockSpec(
                        (1, 1, _FF // 2, _D),
                        lambda t, j=j: (j, jnp.clip(t - 1, 0, _E - 1), 1, 0)),
                ],
                out_specs=pl.BlockSpec((_B, _D), lambda t: (0, 0)),
                out_shape=sd((_B, _D), f32),
                scratch_shapes=[
                    pltpu.VMEM((_S, _D), f32),
                    pltpu.VMEM((_B, _S), f32),
                ],
                compiler_params=_PARAMS)(
                    h, ln1r, attn_v, attn_o, ln2r, router_W,
                    moe_wi, moe_wi, moe_wo, moe_wo)
        else:
            h = pl.pallas_call(
                _dense_kernel,
                grid=(5,),
                in_specs=[
                    pl.BlockSpec((_B, _D), lambda t: (0, 0)),
                    pl.BlockSpec((1, 1, _D), lambda t, i=i: (i, 0, 0)),
                    pl.BlockSpec((1, _D, _D), lambda t, i=i: (i, 0, 0)),
                    pl.BlockSpec((1, _D, _D), lambda t, i=i: (i, 0, 0)),
                    pl.BlockSpec((1, 1, _D), lambda t, i=i: (i, 0, 0)),
                    pl.BlockSpec((1, _D, _D),
                                 lambda t, j=j: (j, 0, jnp.clip(t - 1, 0, 3))),
                    pl.BlockSpec((1, _D, _D),
                                 lambda t, j=j: (j, jnp.clip(t - 1, 0, 3), 0)),
                ],
                out_specs=pl.BlockSpec((_B, _D), lambda t: (0, 0)),
                out_shape=sd((_B, _D), f32),
                scratch_shapes=[
                    pltpu.VMEM((_B, _D), f32),
                ],
                compiler_params=_PARAMS)(
                    h, ln1r, attn_v, attn_o, ln2r, ffn_wi, ffn_wo)

    out = pl.pallas_call(
        _final_kernel,
        out_shape=sd((_B, 10), f32),
        compiler_params=_PARAMS)(
            h, final_ln.reshape(1, _D), fc_W, fc_b.reshape(1, 10))
    return out
